# baseline (speedup 1.0000x reference)
import functools

import jax
import jax.numpy as jnp
from jax import lax
from jax.experimental import pallas as pl
from jax.experimental.pallas import tpu as pltpu
from jax.experimental.pallas import tpu_sc as plsc

N_WORD = 1000
N_PHONE = 64
PAD_ROWS = 1024
ROWS_PER_TILE = 64
TAIL_ROWS = N_WORD - 15 * ROWS_PER_TILE
NC = 2
NS = 16
NW = NC * NS
B = 4096 * 50
BPW = B // NW
CHUNK = 400
NBUF = 2
NCHUNK = BPW // CHUNK
UNROLL = 16


def _body(x_hbm, counts_hbm, out_hbm, table_hbm,
          rowbuf, table_t, idx_v, bufs, ssem, isem, tsem):
    c = lax.axis_index("c")
    s = lax.axis_index("s")

    w = s * NC + c
    base = w * BPW
    idx_cp = pltpu.async_copy(x_hbm.at[pl.ds(base, BPW)], idx_v, isem)

    base_row = s * ROWS_PER_TILE

    @pl.when(s < NS - 1)
    def _():
        pltpu.sync_copy(counts_hbm.at[pl.ds(base_row, ROWS_PER_TILE), :], rowbuf)

    @pl.when(s == NS - 1)
    def _():
        pltpu.sync_copy(
            counts_hbm.at[pl.ds(N_WORD - TAIL_ROWS, TAIL_ROWS), :],
            rowbuf.at[pl.ds(0, TAIL_ROWS), :],
        )

    lanes = lax.iota(jnp.int32, 16)
    perms = [jnp.bitwise_xor(lanes, k) for k in (8, 4, 2, 1)]
    gdn = lax.GatherDimensionNumbers(
        offset_dims=(), collapsed_slice_dims=(0,), start_index_map=(0,)
    )

    def shuffle(v, perm):
        return lax.gather(
            v, perm[:, None], gdn, slice_sizes=(1,),
            mode=lax.GatherScatterMode.PROMISE_IN_BOUNDS,
        )

    def norm_row(i, carry):
        v0 = rowbuf[i, pl.ds(0, 16)]
        v1 = rowbuf[i, pl.ds(16, 16)]
        v2 = rowbuf[i, pl.ds(32, 16)]
        v3 = rowbuf[i, pl.ds(48, 16)]
        t = (v0 + v1) + (v2 + v3)
        for perm in perms:
            t = t + shuffle(t, perm)
        inv = jnp.where(t > 0.0, 1.0 / t, 1.0)
        rowbuf[i, pl.ds(0, 16)] = v0 * inv
        rowbuf[i, pl.ds(16, 16)] = v1 * inv
        rowbuf[i, pl.ds(32, 16)] = v2 * inv
        rowbuf[i, pl.ds(48, 16)] = v3 * inv
        return carry

    lax.fori_loop(0, ROWS_PER_TILE, norm_row, 0)
    pltpu.sync_copy(rowbuf, table_hbm.at[pl.ds(base_row, ROWS_PER_TILE), :])
    plsc.subcore_barrier()

    tbl_cp = pltpu.async_copy(table_hbm, table_t, tsem)
    idx_cp.wait()
    tbl_cp.wait()

    def make_row_copy(goff, buf):
        def row_copy(i, carry):
            r0 = i * UNROLL
            xiv = idx_v[pl.ds(goff + r0, UNROLL)]
            for u in range(UNROLL):
                r = r0 + u
                xi = xiv[u]
                buf[r, pl.ds(0, 16)] = table_t[xi, pl.ds(0, 16)]
                buf[r, pl.ds(16, 16)] = table_t[xi, pl.ds(16, 16)]
                buf[r, pl.ds(32, 16)] = table_t[xi, pl.ds(32, 16)]
                buf[r, pl.ds(48, 16)] = table_t[xi, pl.ds(48, 16)]
            return carry
        return row_copy

    scp = [None] * NBUF
    for g in range(NCHUNK):
        b = g % NBUF
        if scp[b] is not None:
            scp[b].wait()
            scp[b] = None
        lax.fori_loop(0, CHUNK // UNROLL, make_row_copy(g * CHUNK, bufs[b]), 0)
        scp[b] = pltpu.async_copy(
            bufs[b], out_hbm.at[pl.ds(base + g * CHUNK, CHUNK), :], ssem[b]
        )
    for b in range(NBUF):
        if scp[b] is not None:
            scp[b].wait()


@jax.jit
def _run(x_flat, pron_counts):
    mesh = plsc.VectorSubcoreMesh(core_axis_name="c", subcore_axis_name="s")
    f = pl.kernel(
        _body,
        out_type=(
            jax.ShapeDtypeStruct((B, N_PHONE), jnp.float32),
            jax.ShapeDtypeStruct((PAD_ROWS, N_PHONE), jnp.float32),
        ),
        mesh=mesh,
        scratch_types=[
            pltpu.VMEM((ROWS_PER_TILE, N_PHONE), jnp.float32),
            pltpu.VMEM((PAD_ROWS, N_PHONE), jnp.float32),
            pltpu.VMEM((BPW,), jnp.int32),
            [pltpu.VMEM((CHUNK, N_PHONE), jnp.float32)] * NBUF,
            [pltpu.SemaphoreType.DMA] * NBUF,
            pltpu.SemaphoreType.DMA,
            pltpu.SemaphoreType.DMA,
        ],
        compiler_params=pltpu.CompilerParams(use_tc_tiling_on_sc=False),
    )
    out, _ = f(x_flat, pron_counts)
    return out


def kernel(x, pron_counts):
    out = _run(x.reshape(-1), pron_counts)
    return out.reshape(x.shape[0], x.shape[1], N_PHONE)


# speedup vs baseline: 1.4834x; 1.4834x over previous
import functools

import jax
import jax.numpy as jnp
from jax import lax
from jax.experimental import pallas as pl
from jax.experimental.pallas import tpu as pltpu
from jax.experimental.pallas import tpu_sc as plsc

N_WORD = 1000
N_PHONE = 64
PAD_ROWS = 1024
ROWS_PER_TILE = 64
TAIL_ROWS = N_WORD - 15 * ROWS_PER_TILE
NC = 2
NS = 16
NW = NC * NS
B = 4096 * 50
BPW = B // NW
CHUNK = 200
NBUF = 8
NCHUNK = BPW // CHUNK


def _body(x_hbm, counts_hbm, out_hbm, rowbuf, table_sh, idx_v, bufs, gsem, ssem, isem):
    c = lax.axis_index("c")
    s = lax.axis_index("s")

    w = s * NC + c
    base = w * BPW
    idx_cp = pltpu.async_copy(x_hbm.at[pl.ds(base, BPW)], idx_v, isem)

    base_row = s * ROWS_PER_TILE

    @pl.when(s < NS - 1)
    def _():
        pltpu.sync_copy(counts_hbm.at[pl.ds(base_row, ROWS_PER_TILE), :], rowbuf)

    @pl.when(s == NS - 1)
    def _():
        pltpu.sync_copy(
            counts_hbm.at[pl.ds(N_WORD - TAIL_ROWS, TAIL_ROWS), :],
            rowbuf.at[pl.ds(0, TAIL_ROWS), :],
        )

    lanes = lax.iota(jnp.int32, 16)
    perms = [jnp.bitwise_xor(lanes, k) for k in (8, 4, 2, 1)]
    gdn = lax.GatherDimensionNumbers(
        offset_dims=(), collapsed_slice_dims=(0,), start_index_map=(0,)
    )

    def shuffle(v, perm):
        return lax.gather(
            v, perm[:, None], gdn, slice_sizes=(1,),
            mode=lax.GatherScatterMode.PROMISE_IN_BOUNDS,
        )

    def norm_row(i, carry):
        v0 = rowbuf[i, pl.ds(0, 16)]
        v1 = rowbuf[i, pl.ds(16, 16)]
        v2 = rowbuf[i, pl.ds(32, 16)]
        v3 = rowbuf[i, pl.ds(48, 16)]
        t = (v0 + v1) + (v2 + v3)
        for perm in perms:
            t = t + shuffle(t, perm)
        inv = jnp.where(t > 0.0, 1.0 / t, 1.0)
        rowbuf[i, pl.ds(0, 16)] = v0 * inv
        rowbuf[i, pl.ds(16, 16)] = v1 * inv
        rowbuf[i, pl.ds(32, 16)] = v2 * inv
        rowbuf[i, pl.ds(48, 16)] = v3 * inv
        return carry

    lax.fori_loop(0, ROWS_PER_TILE, norm_row, 0)
    pltpu.sync_copy(rowbuf, table_sh.at[pl.ds(base_row, ROWS_PER_TILE), :])
    plsc.subcore_barrier()

    idx_cp.wait()

    def start_gather(g):
        b = g % NBUF
        return pltpu.async_copy(
            table_sh.at[idx_v.at[pl.ds(g * CHUNK, CHUNK)]], bufs[b], gsem[b]
        )

    def start_scatter(g):
        b = g % NBUF
        return pltpu.async_copy(
            bufs[b], out_hbm.at[pl.ds(base + g * CHUNK, CHUNK), :], ssem[b]
        )

    gcp = [None] * NBUF
    scp = [None] * NBUF
    for g in range(NBUF - 1):
        gcp[g % NBUF] = start_gather(g)
    for g in range(NCHUNK):
        b = g % NBUF
        nxt = g + NBUF - 1
        if nxt < NCHUNK:
            nb = nxt % NBUF
            if scp[nb] is not None:
                scp[nb].wait()
                scp[nb] = None
            gcp[nb] = start_gather(nxt)
        gcp[b].wait()
        scp[b] = start_scatter(g)
    for b in range(NBUF):
        if scp[b] is not None:
            scp[b].wait()


@jax.jit
def _run(x_flat, pron_counts):
    mesh = plsc.VectorSubcoreMesh(core_axis_name="c", subcore_axis_name="s")
    f = pl.kernel(
        _body,
        out_type=jax.ShapeDtypeStruct((B, N_PHONE), jnp.float32),
        mesh=mesh,
        scratch_types=[
            pltpu.VMEM((ROWS_PER_TILE, N_PHONE), jnp.float32),
            pltpu.VMEM_SHARED((PAD_ROWS, N_PHONE), jnp.float32),
            pltpu.VMEM((BPW,), jnp.int32),
            [pltpu.VMEM((CHUNK, N_PHONE), jnp.float32)] * NBUF,
            [pltpu.SemaphoreType.DMA] * NBUF,
            [pltpu.SemaphoreType.DMA] * NBUF,
            pltpu.SemaphoreType.DMA,
        ],
        compiler_params=pltpu.CompilerParams(use_tc_tiling_on_sc=False),
    )
    return f(x_flat, pron_counts)


def kernel(x, pron_counts):
    out = _run(x.reshape(-1), pron_counts)
    return out.reshape(x.shape[0], x.shape[1], N_PHONE)
